# Initial kernel scaffold; baseline (speedup 1.0000x reference)
import jax
import jax.numpy as jnp
from jax.experimental import pallas as pl


def kernel(x, pe_weight):
    raise NotImplementedError("write your pallas kernel here")


# speedup vs baseline: 1.7276x; 1.7276x over previous
import jax
import jax.numpy as jnp
from jax.experimental import pallas as pl
from jax.experimental.pallas import tpu as pltpu

_BLOCK_ROWS = 512


def _add_pe_kernel(x_ref, pe_ref, o_ref):
    o_ref[...] = x_ref[...] + pe_ref[...][None, :, :]


def kernel(x, pe_weight):
    batch, seq_len, embed_dim = x.shape
    pe = pe_weight[:seq_len]
    grid = (seq_len // _BLOCK_ROWS,)
    return pl.pallas_call(
        _add_pe_kernel,
        grid=grid,
        in_specs=[
            pl.BlockSpec((batch, _BLOCK_ROWS, embed_dim), lambda i: (0, i, 0)),
            pl.BlockSpec((_BLOCK_ROWS, embed_dim), lambda i: (i, 0)),
        ],
        out_specs=pl.BlockSpec((batch, _BLOCK_ROWS, embed_dim), lambda i: (0, i, 0)),
        out_shape=jax.ShapeDtypeStruct(x.shape, x.dtype),
        compiler_params=pltpu.CompilerParams(
            dimension_semantics=("arbitrary",),
        ),
    )(x, pe)
